# baseline (speedup 1.0000x reference)
import functools

import jax
import jax.numpy as jnp
from jax import lax
from jax.experimental import pallas as pl
from jax.experimental.pallas import tpu as pltpu
from jax.experimental.pallas import tpu_sc as plsc

_INFO = plsc.get_sparse_core_info()
_NC = _INFO.num_cores
_NS = _INFO.num_subcores
_NW = _NC * _NS

_CHUNK = 128


def _make_lookup(vocab, dim, dim_pad, n_rows):
    assert n_rows % (_NW * _CHUNK) == 0
    rows_per_w = n_rows // _NW
    n_chunks = rows_per_w // _CHUNK
    mesh = plsc.VectorSubcoreMesh(core_axis_name="c", subcore_axis_name="s")

    n_vec = dim_pad // 16

    @functools.partial(
        pl.kernel,
        mesh=mesh,
        out_type=jax.ShapeDtypeStruct((n_rows * dim,), jnp.float32),
        scratch_types=[
            pltpu.VMEM((_CHUNK,), jnp.int32),
            pltpu.VMEM((_CHUNK, dim_pad), jnp.float32),
            pltpu.VMEM((_CHUNK * dim + 16,), jnp.float32),
            pltpu.VMEM_SHARED((vocab, dim_pad), jnp.float32),
            pltpu.SemaphoreType.DMA,
        ],
        compiler_params=pltpu.CompilerParams(use_tc_tiling_on_sc=False),
    )
    def lookup(table_hbm, ids_hbm, out_hbm, idx_v, rows_v, comp_v, table_sp,
               sem):
        s = lax.axis_index("s")
        wid = s * _NC + lax.axis_index("c")
        base = wid * rows_per_w

        @pl.when(s == 0)
        def _stage():
            pltpu.sync_copy(table_hbm, table_sp)

        plsc.subcore_barrier()

        lanes = lax.iota(jnp.int32, 16)

        def chunk_body(i, carry):
            off = base + i * _CHUNK
            pltpu.sync_copy(ids_hbm.at[pl.ds(off, _CHUNK)], idx_v)
            pltpu.async_copy(table_sp.at[idx_v], rows_v, sem).wait()

            def row_body(r, carry2):
                dst0 = r * dim
                for k in range(n_vec):
                    v = rows_v[r, pl.ds(16 * k, 16)]
                    comp_v[pl.ds(dst0 + 16 * k, 16)] = v
                return carry2

            lax.fori_loop(0, _CHUNK, row_body, 0)
            pltpu.sync_copy(comp_v.at[pl.ds(0, _CHUNK * dim)],
                            out_hbm.at[pl.ds(off * dim, _CHUNK * dim)])
            return carry

        lax.fori_loop(0, n_chunks, chunk_body, 0)

    return lookup


def kernel(input_ids, table):
    b, l = input_ids.shape
    vocab, dim = table.shape
    dim_pad = dim + (-dim) % 8
    table_p = jnp.pad(table, ((0, 0), (0, dim_pad - dim)))
    ids_flat = input_ids.reshape(b * l).astype(jnp.int32)
    out = _make_lookup(vocab, dim, dim_pad, b * l)(table_p, ids_flat)
    return out.reshape(b, l, dim)


# speedup vs baseline: 1.3717x; 1.3717x over previous
import functools

import jax
import jax.numpy as jnp
from jax import lax
from jax.experimental import pallas as pl
from jax.experimental.pallas import tpu as pltpu
from jax.experimental.pallas import tpu_sc as plsc

_INFO = plsc.get_sparse_core_info()
_NC = _INFO.num_cores
_NS = _INFO.num_subcores
_NW = _NC * _NS

_CHUNK = 128


def _make_lookup(vocab, dim, dim_pad, n_rows):
    assert n_rows % (_NW * 2 * _CHUNK) == 0
    rows_per_w = n_rows // _NW
    n_chunks = rows_per_w // _CHUNK
    mesh = plsc.VectorSubcoreMesh(core_axis_name="c", subcore_axis_name="s")

    @functools.partial(
        pl.kernel,
        mesh=mesh,
        out_type=jax.ShapeDtypeStruct((n_rows, dim_pad), jnp.float32),
        scratch_types=[
            pltpu.VMEM((_CHUNK,), jnp.int32),
            pltpu.VMEM((_CHUNK,), jnp.int32),
            pltpu.VMEM((_CHUNK, dim_pad), jnp.float32),
            pltpu.VMEM((_CHUNK, dim_pad), jnp.float32),
            pltpu.VMEM_SHARED((vocab, dim_pad), jnp.float32),
            pltpu.SemaphoreType.DMA,
            pltpu.SemaphoreType.DMA,
            pltpu.SemaphoreType.DMA,
            pltpu.SemaphoreType.DMA,
        ],
        compiler_params=pltpu.CompilerParams(use_tc_tiling_on_sc=False),
    )
    def lookup(table_hbm, ids_hbm, out_hbm, idx0, idx1, rows0, rows1,
               table_sp, gsem0, gsem1, wsem0, wsem1):
        s = lax.axis_index("s")
        wid = s * _NC + lax.axis_index("c")
        base = wid * rows_per_w

        @pl.when(s == 0)
        def _stage():
            pltpu.sync_copy(table_hbm, table_sp)

        plsc.subcore_barrier()

        bufs = ((idx0, rows0, gsem0, wsem0), (idx1, rows1, gsem1, wsem1))

        def _half(j, i, buf_a, buf_b, first, last):
            idx_a, rows_a, gsem_a, wsem_a = buf_a
            idx_b, rows_b, gsem_b, wsem_b = buf_b
            off = base + i * _CHUNK
            pltpu.make_async_copy(table_sp.at[idx_a], rows_a, gsem_a).wait()
            pltpu.async_copy(rows_a, out_hbm.at[pl.ds(off, _CHUNK)], wsem_a)

            def _prefetch():
                if not first:
                    prev = off - _CHUNK
                    pltpu.make_async_copy(
                        rows_b, out_hbm.at[pl.ds(prev, _CHUNK)], wsem_b
                    ).wait()
                pltpu.sync_copy(ids_hbm.at[pl.ds(off + _CHUNK, _CHUNK)], idx_b)
                pltpu.async_copy(table_sp.at[idx_b], rows_b, gsem_b)

            if last:
                pl.when(j < n_chunks // 2 - 1)(_prefetch)
            else:
                _prefetch()

        pltpu.sync_copy(ids_hbm.at[pl.ds(base, _CHUNK)], idx0)
        pltpu.async_copy(table_sp.at[idx0], rows0, gsem0)

        def pair_body(j, carry):
            i0 = 2 * j
            _half(j, i0, bufs[0], bufs[1], first=False, last=False)
            _half(j, i0 + 1, bufs[1], bufs[0], first=False, last=True)
            return carry

        _half(0, 0, bufs[0], bufs[1], first=True, last=False)
        _half(0, 1, bufs[1], bufs[0], first=False, last=False)
        lax.fori_loop(1, n_chunks // 2, pair_body, 0)

        end0 = base + (n_chunks - 2) * _CHUNK
        end1 = base + (n_chunks - 1) * _CHUNK
        pltpu.make_async_copy(
            rows0, out_hbm.at[pl.ds(end0, _CHUNK)], wsem0).wait()
        pltpu.make_async_copy(
            rows1, out_hbm.at[pl.ds(end1, _CHUNK)], wsem1).wait()

    return lookup


def kernel(input_ids, table):
    b, l = input_ids.shape
    vocab, dim = table.shape
    dim_pad = dim + (-dim) % 8
    table_p = jnp.pad(table, ((0, 0), (0, dim_pad - dim)))
    ids_flat = input_ids.reshape(b * l).astype(jnp.int32)
    out = _make_lookup(vocab, dim, dim_pad, b * l)(table_p, ids_flat)
    return out[:, :dim].reshape(b, l, dim)


# speedup vs baseline: 1.3970x; 1.0185x over previous
import functools

import jax
import jax.numpy as jnp
from jax import lax
from jax.experimental import pallas as pl
from jax.experimental.pallas import tpu as pltpu
from jax.experimental.pallas import tpu_sc as plsc

_INFO = plsc.get_sparse_core_info()
_NC = _INFO.num_cores
_NS = _INFO.num_subcores
_NW = _NC * _NS

_CHUNK = 160


def _make_lookup(vocab, dim, dim_pad, n_rows):
    assert n_rows % (_NW * 2 * _CHUNK) == 0
    rows_per_w = n_rows // _NW
    n_chunks = rows_per_w // _CHUNK
    mesh = plsc.VectorSubcoreMesh(core_axis_name="c", subcore_axis_name="s")

    @functools.partial(
        pl.kernel,
        mesh=mesh,
        out_type=jax.ShapeDtypeStruct((n_rows, dim_pad), jnp.float32),
        scratch_types=[
            pltpu.VMEM((n_rows // _NW,), jnp.int32),
            pltpu.VMEM((_CHUNK, dim_pad), jnp.float32),
            pltpu.VMEM((_CHUNK, dim_pad), jnp.float32),
            pltpu.VMEM_SHARED((vocab, dim_pad), jnp.float32),
            pltpu.SemaphoreType.DMA,
            pltpu.SemaphoreType.DMA,
            pltpu.SemaphoreType.DMA,
            pltpu.SemaphoreType.DMA,
        ],
        compiler_params=pltpu.CompilerParams(use_tc_tiling_on_sc=False),
    )
    def lookup(table_hbm, ids_hbm, out_hbm, idx_all, rows0, rows1,
               table_sp, gsem0, gsem1, wsem0, wsem1):
        s = lax.axis_index("s")
        wid = s * _NC + lax.axis_index("c")
        base = wid * rows_per_w

        @pl.when(s == 0)
        def _stage():
            pltpu.sync_copy(table_hbm, table_sp)

        pltpu.sync_copy(ids_hbm.at[pl.ds(base, rows_per_w)], idx_all)
        plsc.subcore_barrier()

        bufs = ((rows0, gsem0, wsem0), (rows1, gsem1, wsem1))

        def _idx(i):
            return idx_all.at[pl.ds(i * _CHUNK, _CHUNK)]

        def _half(j, i, buf_a, buf_b, first, last):
            rows_a, gsem_a, wsem_a = buf_a
            rows_b, gsem_b, wsem_b = buf_b
            off = base + i * _CHUNK
            pltpu.make_async_copy(table_sp.at[_idx(i)], rows_a, gsem_a).wait()
            pltpu.async_copy(rows_a, out_hbm.at[pl.ds(off, _CHUNK)], wsem_a)

            def _prefetch():
                if not first:
                    prev = off - _CHUNK
                    pltpu.make_async_copy(
                        rows_b, out_hbm.at[pl.ds(prev, _CHUNK)], wsem_b
                    ).wait()
                pltpu.async_copy(table_sp.at[_idx(i + 1)], rows_b, gsem_b)

            if last:
                pl.when(j < n_chunks // 2 - 1)(_prefetch)
            else:
                _prefetch()

        pltpu.async_copy(table_sp.at[_idx(0)], rows0, gsem0)

        def pair_body(j, carry):
            i0 = 2 * j
            _half(j, i0, bufs[0], bufs[1], first=False, last=False)
            _half(j, i0 + 1, bufs[1], bufs[0], first=False, last=True)
            return carry

        _half(0, 0, bufs[0], bufs[1], first=True, last=False)
        _half(0, 1, bufs[1], bufs[0], first=False, last=False)
        lax.fori_loop(1, n_chunks // 2, pair_body, 0)

        end0 = base + (n_chunks - 2) * _CHUNK
        end1 = base + (n_chunks - 1) * _CHUNK
        pltpu.make_async_copy(
            rows0, out_hbm.at[pl.ds(end0, _CHUNK)], wsem0).wait()
        pltpu.make_async_copy(
            rows1, out_hbm.at[pl.ds(end1, _CHUNK)], wsem1).wait()

    return lookup


def kernel(input_ids, table):
    b, l = input_ids.shape
    vocab, dim = table.shape
    dim_pad = dim + (-dim) % 8
    table_p = jnp.pad(table, ((0, 0), (0, dim_pad - dim)))
    ids_flat = input_ids.reshape(b * l).astype(jnp.int32)
    out = _make_lookup(vocab, dim, dim_pad, b * l)(table_p, ids_flat)
    return out[:, :dim].reshape(b, l, dim)
